# baseline (speedup 1.0000x reference)
import jax
import jax.numpy as jnp
from jax.experimental import pallas as pl
from jax.experimental.pallas import tpu as pltpu

DIM = 256
K = 64
PIXEL_SIZE = 50
K_BAN = 10
TEMP = 0.1
BASE_TEMP = 2.0
KNEG = PIXEL_SIZE - K_BAN
M = 4 * 1024
NQ = K * KNEG
TILE = 256
NT = M // TILE
SCALE = -(TEMP / BASE_TEMP)
RSQ = 1.0 / TEMP ** 0.5


def _kernel(feats_ref, off_ref, cc_ref, ccb_ref, xq_ref, b_ref, out_ref,
            nf_ref, lab_ref, labr_ref, ohc_ref, hist_ref, cs_ref, xb_ref,
            t_ref):
    i = pl.program_id(0)

    @pl.when(i == 0)
    def _():
        xb_ref[...] = jax.lax.dot_general(
            xq_ref[...], b_ref[...], (((0,), (0,)), ((), ())),
            preferred_element_type=jnp.float32).astype(jnp.bfloat16)

    @pl.when(i < NT)
    def _phase1():
        f = feats_ref[...]
        nrm = jnp.sqrt(jnp.sum(f * f, axis=1, keepdims=True))
        a = (f * (RSQ / jnp.maximum(nrm, 1e-12))).astype(jnp.bfloat16)
        nf_ref[pl.ds(i * TILE, TILE), :] = a
        o = off_ref[...]
        la = jax.lax.dot_general(o, cc_ref[...], (((1,), (1,)), ((), ())),
                                 preferred_element_type=jnp.float32)
        mx = jnp.max(la, axis=1, keepdims=True)
        col = jax.lax.broadcasted_iota(jnp.int32, la.shape, 1)
        idx = jnp.min(jnp.where(la >= mx, col, K), axis=1, keepdims=True)
        labf = idx.astype(jnp.float32)
        lab_ref[pl.ds(i * TILE, TILE), :] = labf
        labr_ref[:, pl.ds(i * TILE, TILE)] = labf.T
        oh = (idx == jax.lax.broadcasted_iota(jnp.int32, (TILE, K), 1))
        ohf = oh.astype(jnp.float32)
        ohc_ref[pl.ds(i * TILE, TILE), :] = ohf.astype(jnp.bfloat16)
        part = jnp.sum(ohf, axis=0, keepdims=True)
        hist_ref[...] = jnp.where(i == 0, part, hist_ref[...] + part)
        selc = ohf

        l2 = jax.lax.dot_general(a, xq_ref[...], (((1,), (1,)), ((), ())),
                                 preferred_element_type=jnp.float32)
        t2 = jnp.exp(l2.astype(jnp.bfloat16))
        t2b = jnp.dot(t2, b_ref[...], preferred_element_type=jnp.float32)
        s2b = jnp.dot(a, xb_ref[...], preferred_element_type=jnp.float32)
        sum_t2 = jnp.sum(t2b, axis=1, keepdims=True)
        pos_t2 = jnp.sum(selc * t2b, axis=1, keepdims=True)
        neg2 = sum_t2 - pos_t2
        lg2 = jnp.log(t2 + neg2.astype(jnp.bfloat16))
        lg2b = jnp.dot(lg2, b_ref[...], preferred_element_type=jnp.float32)
        num2 = jnp.sum(selc * (s2b - lg2b), axis=1, keepdims=True)
        ppc2_num = jnp.sum(SCALE * num2 / float(KNEG))

        l3 = jax.lax.dot_general(a, ccb_ref[...], (((1,), (1,)), ((), ())),
                                 preferred_element_type=jnp.float32)
        t3 = jnp.exp(l3)
        sum_t3 = jnp.sum(t3, axis=1, keepdims=True)
        pos_t3 = jnp.sum(selc * t3, axis=1, keepdims=True)
        pos_l3 = jnp.sum(selc * l3, axis=1, keepdims=True)
        pcc_num = jnp.sum(SCALE * (pos_l3 - jnp.log(sum_t3)))

        lane = jax.lax.broadcasted_iota(jnp.int32, (1, 128), 1)
        part_v = (jnp.where(lane == 2, ppc2_num, 0.0)
                  + jnp.where(lane == 3, pcc_num, 0.0))
        out_ref[...] = jnp.where(i == 0, part_v, out_ref[...] + part_v)

    @pl.when(i >= NT)
    def _phase2():
        j = i - NT

        @pl.when(i == NT)
        def _():
            cs_ref[...] = jax.lax.dot_general(
                nf_ref[...], ohc_ref[...], (((0,), (0,)), ((), ())),
                preferred_element_type=jnp.float32).astype(jnp.bfloat16)

        a = nf_ref[pl.ds(j * TILE, TILE), :]
        lab_r = lab_ref[pl.ds(j * TILE, TILE), :]
        selc = (lab_r == jax.lax.broadcasted_iota(
            jnp.int32, (TILE, K), 1).astype(jnp.float32)).astype(jnp.float32)

        l1 = jax.lax.dot_general(a, nf_ref[...], (((1,), (1,)), ((), ())),
                                 preferred_element_type=jnp.float32)
        t_ref[...] = jnp.exp(l1.astype(jnp.bfloat16))
        t1b = jnp.dot(t_ref[...], ohc_ref[...],
                      preferred_element_type=jnp.float32)
        s1b = jnp.dot(a, cs_ref[...], preferred_element_type=jnp.float32)
        af = a.astype(jnp.float32)
        lii = jnp.sum(af * af, axis=1, keepdims=True)
        tii = jnp.exp(lii.astype(jnp.bfloat16).astype(jnp.float32))
        tii = tii.astype(jnp.bfloat16).astype(jnp.float32)
        sum_t = jnp.sum(t1b, axis=1, keepdims=True)
        pos_t = jnp.sum(selc * t1b, axis=1, keepdims=True)
        neg1 = sum_t - pos_t + tii
        eqm = lab_r == labr_ref[...]
        lgv = jnp.log(t_ref[...] + neg1.astype(jnp.bfloat16))
        lgs = jnp.sum(jnp.where(eqm, lgv.astype(jnp.float32), 0.0),
                      axis=1, keepdims=True)
        sum_pl = jnp.sum(selc * s1b, axis=1, keepdims=True) - lii
        sum_lg = lgs - jnp.log(tii + neg1)
        cnt = jnp.sum(selc * hist_ref[...], axis=1, keepdims=True) - 1.0
        mlpp1 = (sum_pl - sum_lg) / jnp.maximum(cnt, 1.0)
        valid = (cnt > 0.0).astype(jnp.float32)
        ppc_num = jnp.sum(valid * SCALE * mlpp1)
        ppc_val = jnp.sum(valid)

        lane = jax.lax.broadcasted_iota(jnp.int32, (1, 128), 1)
        part_v = (jnp.where(lane == 0, ppc_num, 0.0)
                  + jnp.where(lane == 1, ppc_val, 0.0))
        out_ref[...] = out_ref[...] + part_v


def kernel(feats, off_feats, cluster_center, point_queue):
    feats2 = feats.reshape(M, DIM)
    off2 = off_feats.reshape(M, DIM)
    xq = (point_queue[:, :KNEG, :].reshape(NQ, DIM) * RSQ).astype(jnp.bfloat16)
    ccb = (cluster_center * RSQ).astype(jnp.bfloat16)
    bmat = (jnp.arange(NQ, dtype=jnp.int32)[:, None] // KNEG
            == jnp.arange(K, dtype=jnp.int32)[None, :]).astype(jnp.bfloat16)

    def clamp(i):
        return (jnp.minimum(i, NT - 1), 0)

    parts = pl.pallas_call(
        _kernel,
        grid=(2 * NT,),
        in_specs=[
            pl.BlockSpec((TILE, DIM), clamp),
            pl.BlockSpec((TILE, DIM), clamp),
            pl.BlockSpec((K, DIM), lambda i: (0, 0)),
            pl.BlockSpec((K, DIM), lambda i: (0, 0)),
            pl.BlockSpec((NQ, DIM), lambda i: (0, 0)),
            pl.BlockSpec((NQ, K), lambda i: (0, 0)),
        ],
        out_specs=pl.BlockSpec((1, 128), lambda i: (0, 0)),
        out_shape=jax.ShapeDtypeStruct((1, 128), jnp.float32),
        scratch_shapes=[
            pltpu.VMEM((M, DIM), jnp.bfloat16),
            pltpu.VMEM((M, 1), jnp.float32),
            pltpu.VMEM((1, M), jnp.float32),
            pltpu.VMEM((M, K), jnp.bfloat16),
            pltpu.VMEM((1, K), jnp.float32),
            pltpu.VMEM((DIM, K), jnp.bfloat16),
            pltpu.VMEM((DIM, K), jnp.bfloat16),
            pltpu.VMEM((TILE, M), jnp.bfloat16),
        ],
    )(feats2, off2, cluster_center, ccb, xq, bmat)

    p = parts[0]
    loss_ppc = p[0] / jnp.maximum(p[1], 1.0)
    loss_ppc2 = p[2] / float(M)
    loss_pcc = p[3] / float(M)
    return loss_ppc + loss_ppc2 + loss_pcc


# speedup vs baseline: 1.0396x; 1.0396x over previous
import jax
import jax.numpy as jnp
from jax.experimental import pallas as pl
from jax.experimental.pallas import tpu as pltpu

DIM = 256
K = 64
PIXEL_SIZE = 50
K_BAN = 10
TEMP = 0.1
BASE_TEMP = 2.0
KNEG = PIXEL_SIZE - K_BAN
M = 4 * 1024
NQ = K * KNEG
TILE = 256
NT = M // TILE
SCALE = -(TEMP / BASE_TEMP)
RSQ = 1.0 / TEMP ** 0.5


def _kernel(feats_ref, off_ref, cc_ref, ccb_ref, xq_ref, b_ref, out_ref,
            nf_ref, lab_ref, ohc_ref, hist_ref, cs_ref, xb_ref, t_ref,
            lg_ref):
    i = pl.program_id(0)

    @pl.when(i == 0)
    def _():
        xb_ref[...] = jax.lax.dot_general(
            xq_ref[...], b_ref[...], (((0,), (0,)), ((), ())),
            preferred_element_type=jnp.float32).astype(jnp.bfloat16)

    @pl.when(i < NT)
    def _phase1():
        f = feats_ref[...]
        nrm = jnp.sqrt(jnp.sum(f * f, axis=1, keepdims=True))
        a = (f * (RSQ / jnp.maximum(nrm, 1e-12))).astype(jnp.bfloat16)
        nf_ref[pl.ds(i * TILE, TILE), :] = a
        o = off_ref[...]
        la = jax.lax.dot_general(o, cc_ref[...], (((1,), (1,)), ((), ())),
                                 preferred_element_type=jnp.float32)
        mx = jnp.max(la, axis=1, keepdims=True)
        col = jax.lax.broadcasted_iota(jnp.int32, la.shape, 1)
        idx = jnp.min(jnp.where(la >= mx, col, K), axis=1, keepdims=True)
        labf = idx.astype(jnp.float32)
        lab_ref[pl.ds(i * TILE, TILE), :] = labf
        oh = (idx == jax.lax.broadcasted_iota(jnp.int32, (TILE, K), 1))
        ohf = oh.astype(jnp.float32)
        ohc_ref[pl.ds(i * TILE, TILE), :] = ohf.astype(jnp.bfloat16)
        part = jnp.sum(ohf, axis=0, keepdims=True)
        hist_ref[...] = jnp.where(i == 0, part, hist_ref[...] + part)
        selc = ohf

        l2 = jax.lax.dot_general(a, xq_ref[...], (((1,), (1,)), ((), ())),
                                 preferred_element_type=jnp.float32)
        t2 = jnp.exp(l2).astype(jnp.bfloat16)
        t2b = jnp.dot(t2, b_ref[...], preferred_element_type=jnp.float32)
        s2b = jnp.dot(a, xb_ref[...], preferred_element_type=jnp.float32)
        sum_t2 = jnp.sum(t2b, axis=1, keepdims=True)
        pos_t2 = jnp.sum(selc * t2b, axis=1, keepdims=True)
        neg2 = sum_t2 - pos_t2
        lg2 = jnp.log(t2 + neg2.astype(jnp.bfloat16))
        lg2b = jnp.dot(lg2, b_ref[...], preferred_element_type=jnp.float32)
        num2 = jnp.sum(selc * (s2b - lg2b), axis=1, keepdims=True)
        ppc2_num = jnp.sum(SCALE * num2 / float(KNEG))

        l3 = jax.lax.dot_general(a, ccb_ref[...], (((1,), (1,)), ((), ())),
                                 preferred_element_type=jnp.float32)
        t3 = jnp.exp(l3)
        sum_t3 = jnp.sum(t3, axis=1, keepdims=True)
        pos_t3 = jnp.sum(selc * t3, axis=1, keepdims=True)
        pos_l3 = jnp.sum(selc * l3, axis=1, keepdims=True)
        pcc_num = jnp.sum(SCALE * (pos_l3 - jnp.log(sum_t3)))

        lane = jax.lax.broadcasted_iota(jnp.int32, (1, 128), 1)
        part_v = (jnp.where(lane == 2, ppc2_num, 0.0)
                  + jnp.where(lane == 3, pcc_num, 0.0))
        out_ref[...] = jnp.where(i == 0, part_v, out_ref[...] + part_v)

    @pl.when(i >= NT)
    def _phase2():
        j = i - NT

        @pl.when(i == NT)
        def _():
            cs_ref[...] = jax.lax.dot_general(
                nf_ref[...], ohc_ref[...], (((0,), (0,)), ((), ())),
                preferred_element_type=jnp.float32).astype(jnp.bfloat16)

        a = nf_ref[pl.ds(j * TILE, TILE), :]
        lab_r = lab_ref[pl.ds(j * TILE, TILE), :]
        selc = (lab_r == jax.lax.broadcasted_iota(
            jnp.int32, (TILE, K), 1).astype(jnp.float32)).astype(jnp.float32)

        l1 = jax.lax.dot_general(a, nf_ref[...], (((1,), (1,)), ((), ())),
                                 preferred_element_type=jnp.float32)
        t_ref[...] = jnp.exp(l1).astype(jnp.bfloat16)
        t1b = jnp.dot(t_ref[...], ohc_ref[...],
                      preferred_element_type=jnp.float32)
        s1b = jnp.dot(a, cs_ref[...], preferred_element_type=jnp.float32)
        af = a.astype(jnp.float32)
        lii = jnp.sum(af * af, axis=1, keepdims=True)
        tii = jnp.exp(lii).astype(jnp.bfloat16).astype(jnp.float32)
        sum_t = jnp.sum(t1b, axis=1, keepdims=True)
        pos_t = jnp.sum(selc * t1b, axis=1, keepdims=True)
        neg1 = sum_t - pos_t + tii
        lg_ref[...] = jnp.log(t_ref[...] + neg1.astype(jnp.bfloat16))
        lg1b = jnp.dot(lg_ref[...], ohc_ref[...],
                       preferred_element_type=jnp.float32)
        sum_pl = jnp.sum(selc * s1b, axis=1, keepdims=True) - lii
        sum_lg = (jnp.sum(selc * lg1b, axis=1, keepdims=True)
                  - jnp.log(tii + neg1))
        cnt = jnp.sum(selc * hist_ref[...], axis=1, keepdims=True) - 1.0
        mlpp1 = (sum_pl - sum_lg) / jnp.maximum(cnt, 1.0)
        valid = (cnt > 0.0).astype(jnp.float32)
        ppc_num = jnp.sum(valid * SCALE * mlpp1)
        ppc_val = jnp.sum(valid)

        lane = jax.lax.broadcasted_iota(jnp.int32, (1, 128), 1)
        part_v = (jnp.where(lane == 0, ppc_num, 0.0)
                  + jnp.where(lane == 1, ppc_val, 0.0))
        out_ref[...] = out_ref[...] + part_v


def kernel(feats, off_feats, cluster_center, point_queue):
    feats2 = feats.reshape(M, DIM)
    off2 = off_feats.reshape(M, DIM)
    xq = (point_queue[:, :KNEG, :].reshape(NQ, DIM) * RSQ).astype(jnp.bfloat16)
    ccb = (cluster_center * RSQ).astype(jnp.bfloat16)
    bmat = (jnp.arange(NQ, dtype=jnp.int32)[:, None] // KNEG
            == jnp.arange(K, dtype=jnp.int32)[None, :]).astype(jnp.bfloat16)

    def clamp(i):
        return (jnp.minimum(i, NT - 1), 0)

    parts = pl.pallas_call(
        _kernel,
        grid=(2 * NT,),
        in_specs=[
            pl.BlockSpec((TILE, DIM), clamp),
            pl.BlockSpec((TILE, DIM), clamp),
            pl.BlockSpec((K, DIM), lambda i: (0, 0)),
            pl.BlockSpec((K, DIM), lambda i: (0, 0)),
            pl.BlockSpec((NQ, DIM), lambda i: (0, 0)),
            pl.BlockSpec((NQ, K), lambda i: (0, 0)),
        ],
        out_specs=pl.BlockSpec((1, 128), lambda i: (0, 0)),
        out_shape=jax.ShapeDtypeStruct((1, 128), jnp.float32),
        scratch_shapes=[
            pltpu.VMEM((M, DIM), jnp.bfloat16),
            pltpu.VMEM((M, 1), jnp.float32),
            pltpu.VMEM((M, K), jnp.bfloat16),
            pltpu.VMEM((1, K), jnp.float32),
            pltpu.VMEM((DIM, K), jnp.bfloat16),
            pltpu.VMEM((DIM, K), jnp.bfloat16),
            pltpu.VMEM((TILE, M), jnp.bfloat16),
            pltpu.VMEM((TILE, M), jnp.bfloat16),
        ],
    )(feats2, off2, cluster_center, ccb, xq, bmat)

    p = parts[0]
    loss_ppc = p[0] / jnp.maximum(p[1], 1.0)
    loss_ppc2 = p[2] / float(M)
    loss_pcc = p[3] / float(M)
    return loss_ppc + loss_ppc2 + loss_pcc
